# Initial kernel scaffold; baseline (speedup 1.0000x reference)
import jax
import jax.numpy as jnp
from jax.experimental import pallas as pl


def kernel(x_0, batch_0, W, b):
    raise NotImplementedError("write your pallas kernel here")


# speedup vs baseline: 17.7321x; 17.7321x over previous
import functools

import jax
import jax.numpy as jnp
from jax import lax
from jax.experimental import pallas as pl
from jax.experimental.pallas import tpu as pltpu

_N = 50000
_D = 512
_G = 128
_B = 2000
_NB = _N // _B


def _pool_body(batch_ref, x_ref, out_ref):
    i = pl.program_id(0)
    ids = batch_ref[0, 0, :]
    seg = lax.broadcasted_iota(jnp.int32, (_G, _B), 0)
    onehot = jnp.where(seg == ids[None, :], 1.0, 0.0).astype(jnp.float32)
    part = jax.lax.dot_general(
        onehot, x_ref[...],
        dimension_numbers=(((1,), (0,)), ((), ())),
        preferred_element_type=jnp.float32)

    @pl.when(i == 0)
    def _():
        out_ref[...] = part

    @pl.when(i != 0)
    def _():
        out_ref[...] += part


def _proj_body(pooled_ref, w_ref, b_ref, out_ref):
    out_ref[...] = jax.lax.dot_general(
        pooled_ref[...], w_ref[...],
        dimension_numbers=(((1,), (1,)), ((), ())),
        preferred_element_type=jnp.float32) + b_ref[...]


@jax.jit
def kernel(x_0, batch_0, W, b):
    batch3 = batch_0.reshape(_NB, 1, _B)
    pooled = pl.pallas_call(
        _pool_body,
        grid=(_NB,),
        in_specs=[
            pl.BlockSpec((1, 1, _B), lambda i: (i, 0, 0)),
            pl.BlockSpec((_B, _D), lambda i: (i, 0)),
        ],
        out_specs=pl.BlockSpec((_G, _D), lambda i: (0, 0)),
        out_shape=jax.ShapeDtypeStruct((_G, _D), jnp.float32),
    )(batch3, x_0)
    logits = pl.pallas_call(
        _proj_body,
        out_shape=jax.ShapeDtypeStruct((_G, _D), jnp.float32),
    )(pooled, W, b.reshape(1, _D))
    return logits
